# baseline (speedup 1.0000x reference)
import functools

import jax
import jax.numpy as jnp
from jax import lax
from jax.experimental import pallas as pl
from jax.experimental.pallas import tpu as pltpu
from jax.experimental.pallas import tpu_sc as plsc

NN = 10000
DIN = 256
DHID = 256
HALF = 128
NP = 10240
TROW = NP // 16
EB = 128
NTILES = 1250
NTPAD = 1280
NBT = NTPAD // 16
P0 = 40
NCORE = 2
NSUB = 16

_mesh = plsc.VectorSubcoreMesh(core_axis_name="c", subcore_axis_name="s")


@functools.partial(
    pl.kernel,
    mesh=_mesh,
    out_type=jax.ShapeDtypeStruct((NCORE, NP), jnp.float32),
    scratch_types=[
        pltpu.VMEM((NBT, EB), jnp.int32),
        pltpu.VMEM((EB,), jnp.float32),
        pltpu.VMEM((TROW,), jnp.float32),
        pltpu.VMEM_SHARED((NP,), jnp.float32),
        pltpu.SemaphoreType.DMA,
    ],
)
def _deg_sc(e6_hbm, deg_out, didx_v, ones_v, zbuf_v, deg_sh, sem):
    c = lax.axis_index("c")
    s = lax.axis_index("s")
    for i in range(EB // 16):
        ones_v[pl.ds(i * 16, 16)] = jnp.ones((16,), jnp.float32)
    for i in range(TROW // 16):
        zbuf_v[pl.ds(i * 16, 16)] = jnp.zeros((16,), jnp.float32)
    pltpu.sync_copy(e6_hbm.at[1, pl.ds(s * NBT, NBT)], didx_v)
    pltpu.sync_copy(zbuf_v, deg_sh.at[pl.ds(s * TROW, TROW)])
    plsc.subcore_barrier()

    nmine = NBT // 2

    def fire(b, carry):
        pltpu.async_copy(ones_v, deg_sh.at[didx_v.at[2 * b + c]], sem,
                         add=True)
        return carry

    lax.fori_loop(0, nmine, fire, 0)

    def drain(b, carry):
        pltpu.make_async_copy(ones_v, deg_sh.at[didx_v.at[0]], sem).wait()
        return carry

    lax.fori_loop(0, nmine, drain, 0)
    plsc.subcore_barrier()
    pltpu.sync_copy(deg_sh.at[pl.ds(s * TROW, TROW)],
                    deg_out.at[c, pl.ds(s * TROW, TROW)])


@functools.partial(
    pl.kernel,
    mesh=_mesh,
    out_type=jax.ShapeDtypeStruct((NCORE, NP, HALF), jnp.float32),
    scratch_types=[
        pltpu.VMEM((P0, EB), jnp.int32),
        pltpu.VMEM((P0, EB), jnp.int32),
        pltpu.VMEM((2, EB, HALF), jnp.float32),
        pltpu.VMEM_SHARED((NP, HALF), jnp.float32),
        pltpu.SemaphoreType.DMA((2,)),
    ],
)
def _msg_sc(tab3_hbm, e6_hbm, acc_out, sidx_v, didx_v, rows_v, acc_sh, gsem):
    c = lax.axis_index("c")
    s = lax.axis_index("s")

    def zfill(r, carry):
        for j in range(HALF // 16):
            rows_v[0, r, pl.ds(j * 16, 16)] = jnp.zeros((16,), jnp.float32)
        return carry

    lax.fori_loop(0, EB, zfill, 0)
    for k in range(TROW // EB):
        pltpu.sync_copy(rows_v.at[0],
                        acc_sh.at[pl.ds(s * TROW + k * EB, EB)])
    plsc.subcore_barrier()

    def body(b, nbp):
        slot = lax.rem(b, 2)
        nslot = lax.rem(b + 1, 2)

        @pl.when(b + 1 < nbp)
        def _():
            pltpu.async_copy(tab3_hbm.at[c].at[sidx_v.at[b + 1]],
                             rows_v.at[nslot], gsem.at[nslot])

        pltpu.make_async_copy(tab3_hbm.at[c].at[sidx_v.at[0]],
                              rows_v.at[slot], gsem.at[slot]).wait()
        pltpu.sync_copy(rows_v.at[slot], acc_sh.at[didx_v.at[b]], add=True)
        return nbp

    for pb in (0, P0):
        pltpu.sync_copy(e6_hbm.at[0, pl.ds(s * NBT + pb, P0)], sidx_v)
        pltpu.sync_copy(e6_hbm.at[1, pl.ds(s * NBT + pb, P0)], didx_v)
        pltpu.async_copy(tab3_hbm.at[c].at[sidx_v.at[0]], rows_v.at[0],
                         gsem.at[0])
        lax.fori_loop(0, P0, body, P0)
    plsc.subcore_barrier()
    pltpu.sync_copy(acc_sh.at[pl.ds(s * TROW, TROW)],
                    acc_out.at[c, pl.ds(s * TROW, TROW)])


_BN = 512
_NBLK = NP // _BN


def _xws_body(x_ref, w_ref, deg_ref, tab_ref, dis_ref):
    xw = jnp.dot(x_ref[...].astype(jnp.bfloat16),
                 w_ref[...].astype(jnp.bfloat16),
                 preferred_element_type=jnp.float32)
    dt = jnp.transpose(deg_ref[...], (1, 0))
    dis = lax.rsqrt(dt[:, 0:1] + dt[:, 1:2] + 1.0)
    dis_ref[...] = dis
    tab_ref[0] = xw[:, :HALF] * dis
    tab_ref[1] = xw[:, HALF:] * dis


def _xws_tc(x, w, deg2):
    return pl.pallas_call(
        _xws_body,
        grid=(_NBLK,),
        in_specs=[pl.BlockSpec((_BN, DIN), lambda i: (i, 0)),
                  pl.BlockSpec((DIN, DHID), lambda i: (0, 0)),
                  pl.BlockSpec((NCORE, _BN), lambda i: (0, i))],
        out_specs=[pl.BlockSpec((NCORE, _BN, HALF), lambda i: (0, i, 0)),
                   pl.BlockSpec((_BN, 1), lambda i: (i, 0))],
        out_shape=[jax.ShapeDtypeStruct((NCORE, NP, HALF), jnp.float32),
                   jax.ShapeDtypeStruct((NP, 1), jnp.float32)],
    )(x, w, deg2)


def _epi_body(acc_ref, tab_ref, dis_ref, hn_ref, bg_ref, wf_ref, bf_ref,
              wc_ref, bc_ref, o_ref):
    dis = dis_ref[...]
    a = jnp.concatenate(
        [(acc_ref[0] + tab_ref[0]) * dis, (acc_ref[1] + tab_ref[1]) * dis],
        axis=1)
    h = jnp.maximum(a + bg_ref[...], 0.0)
    alpha = 1.0 - jnp.transpose(hn_ref[...], (1, 0))
    ha = alpha * h
    h2 = jnp.maximum(
        jnp.dot(ha, wf_ref[...], preferred_element_type=jnp.float32)
        + bf_ref[...], 0.0)
    out = (jnp.dot(h2, wc_ref[...], preferred_element_type=jnp.float32)
           + bc_ref[...])
    o_ref[...] = jnp.transpose(out, (1, 0))


def _epi_tc(acc, tab, dis, hn, bg, wf, bf, wc, bc):
    nc = wc.shape[1]
    return pl.pallas_call(
        _epi_body,
        grid=(_NBLK,),
        in_specs=[pl.BlockSpec((NCORE, _BN, HALF), lambda i: (0, i, 0)),
                  pl.BlockSpec((NCORE, _BN, HALF), lambda i: (0, i, 0)),
                  pl.BlockSpec((_BN, 1), lambda i: (i, 0)),
                  pl.BlockSpec((1, _BN), lambda i: (0, i)),
                  pl.BlockSpec((1, DHID), lambda i: (0, 0)),
                  pl.BlockSpec((DHID, 64), lambda i: (0, 0)),
                  pl.BlockSpec((1, 64), lambda i: (0, 0)),
                  pl.BlockSpec((64, nc), lambda i: (0, 0)),
                  pl.BlockSpec((1, nc), lambda i: (0, 0))],
        out_specs=pl.BlockSpec((nc, _BN), lambda i: (0, i)),
        out_shape=jax.ShapeDtypeStruct((nc, NN), jnp.float32),
    )(acc, tab, dis, hn, bg, wf, bf, wc, bc)


def kernel(x, edge_index, h_node, W_gcn, b_gcn, W_fus, b_fus, W_cls, b_cls):
    npad = NTPAD - NTILES
    ar = jnp.arange(npad * EB, dtype=jnp.int32).reshape(npad, EB)
    pad = jnp.stack([(ar * 89) % NN, NN + (ar % 16)])
    e6 = jnp.concatenate([edge_index.reshape(2, NTILES, EB), pad], axis=1)
    deg2 = _deg_sc(e6)
    tab, dis = _xws_tc(x, W_gcn, deg2)
    acc = _msg_sc(tab, e6)
    out_t = _epi_tc(acc, tab, dis, h_node.reshape(1, NN),
                    b_gcn.reshape(1, DHID), W_fus, b_fus.reshape(1, 64),
                    W_cls, b_cls.reshape(1, W_cls.shape[1]))
    return out_t.T


# speedup vs baseline: 1.0576x; 1.0576x over previous
import functools

import jax
import jax.numpy as jnp
from jax import lax
from jax.experimental import pallas as pl
from jax.experimental.pallas import tpu as pltpu
from jax.experimental.pallas import tpu_sc as plsc

NN = 10000
DIN = 256
DHID = 256
HALF = 128
NP = 10240
TROW = NP // 16
EB = 128
NTILES = 1250
NTPAD = 1280
NBT = NTPAD // 16
P0 = 40
NCORE = 2
NSUB = 16

_mesh = plsc.VectorSubcoreMesh(core_axis_name="c", subcore_axis_name="s")


@functools.partial(
    pl.kernel,
    mesh=_mesh,
    out_type=jax.ShapeDtypeStruct((NCORE, NP), jnp.float32),
    scratch_types=[
        pltpu.VMEM((NBT, EB), jnp.int32),
        pltpu.VMEM((EB,), jnp.float32),
        pltpu.VMEM((TROW,), jnp.float32),
        pltpu.VMEM_SHARED((NP,), jnp.float32),
        pltpu.SemaphoreType.DMA,
    ],
)
def _deg_sc(e6_hbm, deg_out, didx_v, ones_v, zbuf_v, deg_sh, sem):
    c = lax.axis_index("c")
    s = lax.axis_index("s")
    for i in range(EB // 16):
        ones_v[pl.ds(i * 16, 16)] = jnp.ones((16,), jnp.float32)
    for i in range(TROW // 16):
        zbuf_v[pl.ds(i * 16, 16)] = jnp.zeros((16,), jnp.float32)
    pltpu.sync_copy(e6_hbm.at[1, pl.ds(s * NBT, NBT)], didx_v)
    pltpu.sync_copy(zbuf_v, deg_sh.at[pl.ds(s * TROW, TROW)])
    plsc.subcore_barrier()

    nmine = NBT // 2

    def fire(b, carry):
        pltpu.async_copy(ones_v, deg_sh.at[didx_v.at[2 * b + c]], sem,
                         add=True)
        return carry

    lax.fori_loop(0, nmine, fire, 0)

    def drain(b, carry):
        pltpu.make_async_copy(ones_v, deg_sh.at[didx_v.at[0]], sem).wait()
        return carry

    lax.fori_loop(0, nmine, drain, 0)
    plsc.subcore_barrier()
    pltpu.sync_copy(deg_sh.at[pl.ds(s * TROW, TROW)],
                    deg_out.at[c, pl.ds(s * TROW, TROW)])


@functools.partial(
    pl.kernel,
    mesh=_mesh,
    out_type=jax.ShapeDtypeStruct((NCORE, NP, HALF), jnp.float32),
    scratch_types=[
        pltpu.VMEM((P0, EB), jnp.int32),
        pltpu.VMEM((P0, EB), jnp.int32),
        pltpu.VMEM((2, EB, HALF), jnp.float32),
        pltpu.VMEM_SHARED((NP, HALF), jnp.float32),
        pltpu.SemaphoreType.DMA((2,)),
    ],
)
def _msg_sc(tab3_hbm, e6_hbm, acc_out, sidx_v, didx_v, rows_v, acc_sh, gsem):
    c = lax.axis_index("c")
    s = lax.axis_index("s")

    def zfill(r, carry):
        for j in range(HALF // 16):
            rows_v[0, r, pl.ds(j * 16, 16)] = jnp.zeros((16,), jnp.float32)
        return carry

    lax.fori_loop(0, EB, zfill, 0)
    for k in range(TROW // EB):
        pltpu.sync_copy(rows_v.at[0],
                        acc_sh.at[pl.ds(s * TROW + k * EB, EB)])
    plsc.subcore_barrier()

    def body(b, nbp):
        slot = lax.rem(b, 2)
        nslot = lax.rem(b + 1, 2)

        @pl.when(b + 1 < nbp)
        def _():
            pltpu.async_copy(tab3_hbm.at[c].at[sidx_v.at[b + 1]],
                             rows_v.at[nslot], gsem.at[nslot])

        pltpu.make_async_copy(tab3_hbm.at[c].at[sidx_v.at[0]],
                              rows_v.at[slot], gsem.at[slot]).wait()
        pltpu.sync_copy(rows_v.at[slot], acc_sh.at[didx_v.at[b]], add=True)
        return nbp

    for pb in (0, P0):
        pltpu.sync_copy(e6_hbm.at[0, pl.ds(s * NBT + pb, P0)], sidx_v)
        pltpu.sync_copy(e6_hbm.at[1, pl.ds(s * NBT + pb, P0)], didx_v)
        pltpu.async_copy(tab3_hbm.at[c].at[sidx_v.at[0]], rows_v.at[0],
                         gsem.at[0])
        lax.fori_loop(0, P0, body, P0)
    plsc.subcore_barrier()
    pltpu.sync_copy(acc_sh.at[pl.ds(s * TROW, TROW)],
                    acc_out.at[c, pl.ds(s * TROW, TROW)])


_BN = 1024
_NBLK = NP // _BN


def _xws_body(x_ref, w_ref, deg_ref, tab_ref, dis_ref):
    xw = jnp.dot(x_ref[...].astype(jnp.bfloat16),
                 w_ref[...].astype(jnp.bfloat16),
                 preferred_element_type=jnp.float32)
    dt = jnp.transpose(deg_ref[...], (1, 0))
    dis = lax.rsqrt(dt[:, 0:1] + dt[:, 1:2] + 1.0)
    dis_ref[...] = dis
    tab_ref[0] = xw[:, :HALF] * dis
    tab_ref[1] = xw[:, HALF:] * dis


def _xws_tc(x, w, deg2):
    return pl.pallas_call(
        _xws_body,
        grid=(_NBLK,),
        in_specs=[pl.BlockSpec((_BN, DIN), lambda i: (i, 0)),
                  pl.BlockSpec((DIN, DHID), lambda i: (0, 0)),
                  pl.BlockSpec((NCORE, _BN), lambda i: (0, i))],
        out_specs=[pl.BlockSpec((NCORE, _BN, HALF), lambda i: (0, i, 0)),
                   pl.BlockSpec((_BN, 1), lambda i: (i, 0))],
        out_shape=[jax.ShapeDtypeStruct((NCORE, NP, HALF), jnp.float32),
                   jax.ShapeDtypeStruct((NP, 1), jnp.float32)],
    )(x, w, deg2)


def _epi_body(acc_ref, tab_ref, dis_ref, hn_ref, bg_ref, wf_ref, bf_ref,
              wc_ref, bc_ref, o_ref):
    dis = dis_ref[...]
    a = jnp.concatenate(
        [(acc_ref[0] + tab_ref[0]) * dis, (acc_ref[1] + tab_ref[1]) * dis],
        axis=1)
    h = jnp.maximum(a + bg_ref[...], 0.0)
    alpha = 1.0 - jnp.transpose(hn_ref[...], (1, 0))
    ha = alpha * h
    h2 = jnp.maximum(
        jnp.dot(ha, wf_ref[...], preferred_element_type=jnp.float32)
        + bf_ref[...], 0.0)
    out = (jnp.dot(h2, wc_ref[...], preferred_element_type=jnp.float32)
           + bc_ref[...])
    o_ref[...] = jnp.transpose(out, (1, 0))


def _epi_tc(acc, tab, dis, hn, bg, wf, bf, wc, bc):
    nc = wc.shape[1]
    return pl.pallas_call(
        _epi_body,
        grid=(_NBLK,),
        in_specs=[pl.BlockSpec((NCORE, _BN, HALF), lambda i: (0, i, 0)),
                  pl.BlockSpec((NCORE, _BN, HALF), lambda i: (0, i, 0)),
                  pl.BlockSpec((_BN, 1), lambda i: (i, 0)),
                  pl.BlockSpec((1, _BN), lambda i: (0, i)),
                  pl.BlockSpec((1, DHID), lambda i: (0, 0)),
                  pl.BlockSpec((DHID, 64), lambda i: (0, 0)),
                  pl.BlockSpec((1, 64), lambda i: (0, 0)),
                  pl.BlockSpec((64, nc), lambda i: (0, 0)),
                  pl.BlockSpec((1, nc), lambda i: (0, 0))],
        out_specs=pl.BlockSpec((nc, _BN), lambda i: (0, i)),
        out_shape=jax.ShapeDtypeStruct((nc, NN), jnp.float32),
    )(acc, tab, dis, hn, bg, wf, bf, wc, bc)


def kernel(x, edge_index, h_node, W_gcn, b_gcn, W_fus, b_fus, W_cls, b_cls):
    npad = NTPAD - NTILES
    ar = jnp.arange(npad * EB, dtype=jnp.int32).reshape(npad, EB)
    pad = jnp.stack([(ar * 89) % NN, NN + (ar % 16)])
    e6 = jnp.concatenate([edge_index.reshape(2, NTILES, EB), pad], axis=1)
    deg2 = _deg_sc(e6)
    tab, dis = _xws_tc(x, W_gcn, deg2)
    acc = _msg_sc(tab, e6)
    out_t = _epi_tc(acc, tab, dis, h_node.reshape(1, NN),
                    b_gcn.reshape(1, DHID), W_fus, b_fus.reshape(1, 64),
                    W_cls, b_cls.reshape(1, W_cls.shape[1]))
    return out_t.T


# speedup vs baseline: 1.0936x; 1.0340x over previous
import functools

import jax
import jax.numpy as jnp
from jax import lax
from jax.experimental import pallas as pl
from jax.experimental.pallas import tpu as pltpu
from jax.experimental.pallas import tpu_sc as plsc

NN = 10000
DIN = 256
DHID = 256
HALF = 128
NP = 10240
TROW = NP // 16
EB = 128
NTILES = 1250
NTPAD = 1280
NBT = NTPAD // 16
P0 = 40
NCORE = 2
NSUB = 16

_mesh = plsc.VectorSubcoreMesh(core_axis_name="c", subcore_axis_name="s")


@functools.partial(
    pl.kernel,
    mesh=_mesh,
    out_type=jax.ShapeDtypeStruct((NCORE, NP), jnp.float32),
    scratch_types=[
        pltpu.VMEM((NBT, EB), jnp.int32),
        pltpu.VMEM((EB,), jnp.float32),
        pltpu.VMEM((TROW,), jnp.float32),
        pltpu.VMEM_SHARED((NP,), jnp.float32),
        pltpu.SemaphoreType.DMA,
    ],
)
def _deg_sc(e6_hbm, deg_out, didx_v, ones_v, zbuf_v, deg_sh, sem):
    c = lax.axis_index("c")
    s = lax.axis_index("s")
    for i in range(EB // 16):
        ones_v[pl.ds(i * 16, 16)] = jnp.ones((16,), jnp.float32)
    for i in range(TROW // 16):
        zbuf_v[pl.ds(i * 16, 16)] = jnp.zeros((16,), jnp.float32)
    pltpu.sync_copy(e6_hbm.at[1, pl.ds(s * NBT, NBT)], didx_v)
    pltpu.sync_copy(zbuf_v, deg_sh.at[pl.ds(s * TROW, TROW)])
    plsc.subcore_barrier()

    nmine = NBT // 2

    def fire(b, carry):
        pltpu.async_copy(ones_v, deg_sh.at[didx_v.at[2 * b + c]], sem,
                         add=True)
        return carry

    lax.fori_loop(0, nmine, fire, 0)

    def drain(b, carry):
        pltpu.make_async_copy(ones_v, deg_sh.at[didx_v.at[0]], sem).wait()
        return carry

    lax.fori_loop(0, nmine, drain, 0)
    plsc.subcore_barrier()
    pltpu.sync_copy(deg_sh.at[pl.ds(s * TROW, TROW)],
                    deg_out.at[c, pl.ds(s * TROW, TROW)])


@functools.partial(
    pl.kernel,
    mesh=_mesh,
    out_type=jax.ShapeDtypeStruct((NCORE, NP, HALF), jnp.float32),
    scratch_types=[
        pltpu.VMEM((P0, EB), jnp.int32),
        pltpu.VMEM((P0, EB), jnp.int32),
        pltpu.VMEM((2, EB, HALF), jnp.float32),
        pltpu.VMEM_SHARED((NP, HALF), jnp.float32),
        pltpu.SemaphoreType.DMA((2,)),
    ],
)
def _msg_sc(tab3_hbm, e6_hbm, acc_out, sidx_v, didx_v, rows_v, acc_sh, gsem):
    c = lax.axis_index("c")
    s = lax.axis_index("s")

    def zfill(r, carry):
        for j in range(HALF // 16):
            rows_v[0, r, pl.ds(j * 16, 16)] = jnp.zeros((16,), jnp.float32)
        return carry

    lax.fori_loop(0, EB, zfill, 0)
    for k in range(TROW // EB):
        pltpu.sync_copy(rows_v.at[0],
                        acc_sh.at[pl.ds(s * TROW + k * EB, EB)])
    plsc.subcore_barrier()

    def body(b, nbp):
        slot = lax.rem(b, 2)
        nslot = lax.rem(b + 1, 2)

        @pl.when(b + 1 < nbp)
        def _():
            pltpu.async_copy(tab3_hbm.at[c].at[sidx_v.at[b + 1]],
                             rows_v.at[nslot], gsem.at[nslot])

        pltpu.make_async_copy(tab3_hbm.at[c].at[sidx_v.at[0]],
                              rows_v.at[slot], gsem.at[slot]).wait()
        pltpu.sync_copy(rows_v.at[slot], acc_sh.at[didx_v.at[b]], add=True)
        return nbp

    for pb in (0, P0):
        pltpu.sync_copy(e6_hbm.at[0, pl.ds(s * NBT + pb, P0)], sidx_v)
        pltpu.sync_copy(e6_hbm.at[1, pl.ds(s * NBT + pb, P0)], didx_v)
        pltpu.async_copy(tab3_hbm.at[c].at[sidx_v.at[0]], rows_v.at[0],
                         gsem.at[0])
        lax.fori_loop(0, P0, body, P0)
    plsc.subcore_barrier()
    pltpu.sync_copy(acc_sh.at[pl.ds(s * TROW, TROW)],
                    acc_out.at[c, pl.ds(s * TROW, TROW)])


_BN = 2048
_NBLK = NP // _BN


def _xws_body(x_ref, w_ref, deg_ref, tab_ref, dis_ref):
    xw = jnp.dot(x_ref[...].astype(jnp.bfloat16),
                 w_ref[...].astype(jnp.bfloat16),
                 preferred_element_type=jnp.float32)
    dt = jnp.transpose(deg_ref[...], (1, 0))
    dis = lax.rsqrt(dt[:, 0:1] + dt[:, 1:2] + 1.0)
    dis_ref[...] = dis
    tab_ref[0] = xw[:, :HALF] * dis
    tab_ref[1] = xw[:, HALF:] * dis


def _xws_tc(x, w, deg2):
    return pl.pallas_call(
        _xws_body,
        grid=(_NBLK,),
        in_specs=[pl.BlockSpec((_BN, DIN), lambda i: (i, 0)),
                  pl.BlockSpec((DIN, DHID), lambda i: (0, 0)),
                  pl.BlockSpec((NCORE, _BN), lambda i: (0, i))],
        out_specs=[pl.BlockSpec((NCORE, _BN, HALF), lambda i: (0, i, 0)),
                   pl.BlockSpec((_BN, 1), lambda i: (i, 0))],
        out_shape=[jax.ShapeDtypeStruct((NCORE, NP, HALF), jnp.float32),
                   jax.ShapeDtypeStruct((NP, 1), jnp.float32)],
    )(x, w, deg2)


def _epi_body(acc_ref, tab_ref, dis_ref, hn_ref, bg_ref, wf_ref, bf_ref,
              wc_ref, bc_ref, o_ref):
    dis = dis_ref[...]
    a = jnp.concatenate(
        [(acc_ref[0] + tab_ref[0]) * dis, (acc_ref[1] + tab_ref[1]) * dis],
        axis=1)
    h = jnp.maximum(a + bg_ref[...], 0.0)
    alpha = 1.0 - jnp.transpose(hn_ref[...], (1, 0))
    ha = alpha * h
    h2 = jnp.maximum(
        jnp.dot(ha, wf_ref[...], preferred_element_type=jnp.float32)
        + bf_ref[...], 0.0)
    out = (jnp.dot(h2, wc_ref[...], preferred_element_type=jnp.float32)
           + bc_ref[...])
    o_ref[...] = jnp.transpose(out, (1, 0))


def _epi_tc(acc, tab, dis, hn, bg, wf, bf, wc, bc):
    nc = wc.shape[1]
    return pl.pallas_call(
        _epi_body,
        grid=(_NBLK,),
        in_specs=[pl.BlockSpec((NCORE, _BN, HALF), lambda i: (0, i, 0)),
                  pl.BlockSpec((NCORE, _BN, HALF), lambda i: (0, i, 0)),
                  pl.BlockSpec((_BN, 1), lambda i: (i, 0)),
                  pl.BlockSpec((1, _BN), lambda i: (0, i)),
                  pl.BlockSpec((1, DHID), lambda i: (0, 0)),
                  pl.BlockSpec((DHID, 64), lambda i: (0, 0)),
                  pl.BlockSpec((1, 64), lambda i: (0, 0)),
                  pl.BlockSpec((64, nc), lambda i: (0, 0)),
                  pl.BlockSpec((1, nc), lambda i: (0, 0))],
        out_specs=pl.BlockSpec((nc, _BN), lambda i: (0, i)),
        out_shape=jax.ShapeDtypeStruct((nc, NN), jnp.float32),
    )(acc, tab, dis, hn, bg, wf, bf, wc, bc)


def kernel(x, edge_index, h_node, W_gcn, b_gcn, W_fus, b_fus, W_cls, b_cls):
    npad = NTPAD - NTILES
    ar = jnp.arange(npad * EB, dtype=jnp.int32).reshape(npad, EB)
    pad = jnp.stack([(ar * 89) % NN, NN + (ar % 16)])
    e6 = jnp.concatenate([edge_index.reshape(2, NTILES, EB), pad], axis=1)
    deg2 = _deg_sc(e6)
    tab, dis = _xws_tc(x, W_gcn, deg2)
    acc = _msg_sc(tab, e6)
    out_t = _epi_tc(acc, tab, dis, h_node.reshape(1, NN),
                    b_gcn.reshape(1, DHID), W_fus, b_fus.reshape(1, 64),
                    W_cls, b_cls.reshape(1, W_cls.shape[1]))
    return out_t.T


# speedup vs baseline: 1.1057x; 1.0110x over previous
import functools

import jax
import jax.numpy as jnp
from jax import lax
from jax.experimental import pallas as pl
from jax.experimental.pallas import tpu as pltpu
from jax.experimental.pallas import tpu_sc as plsc

NN = 10000
DIN = 256
DHID = 256
HALF = 128
NP = 10240
TROW = NP // 16
EB = 128
NTILES = 1250
NTPAD = 1280
NBT = NTPAD // 16
P0 = 40
NCORE = 2
NSUB = 16

_mesh = plsc.VectorSubcoreMesh(core_axis_name="c", subcore_axis_name="s")


@functools.partial(
    pl.kernel,
    mesh=_mesh,
    out_type=jax.ShapeDtypeStruct((NCORE, NP), jnp.float32),
    scratch_types=[
        pltpu.VMEM((NBT, EB), jnp.int32),
        pltpu.VMEM((EB,), jnp.float32),
        pltpu.VMEM((TROW,), jnp.float32),
        pltpu.VMEM_SHARED((NP,), jnp.float32),
        pltpu.SemaphoreType.DMA,
    ],
)
def _deg_sc(e6_hbm, deg_out, didx_v, ones_v, zbuf_v, deg_sh, sem):
    c = lax.axis_index("c")
    s = lax.axis_index("s")
    for i in range(EB // 16):
        ones_v[pl.ds(i * 16, 16)] = jnp.ones((16,), jnp.float32)
    for i in range(TROW // 16):
        zbuf_v[pl.ds(i * 16, 16)] = jnp.zeros((16,), jnp.float32)
    pltpu.sync_copy(e6_hbm.at[1, pl.ds(s * NBT, NBT)], didx_v)
    pltpu.sync_copy(zbuf_v, deg_sh.at[pl.ds(s * TROW, TROW)])
    plsc.subcore_barrier()

    nmine = NBT // 2

    def fire(b, carry):
        pltpu.async_copy(ones_v, deg_sh.at[didx_v.at[2 * b + c]], sem,
                         add=True)
        return carry

    lax.fori_loop(0, nmine, fire, 0)

    def drain(b, carry):
        pltpu.make_async_copy(ones_v, deg_sh.at[didx_v.at[0]], sem).wait()
        return carry

    lax.fori_loop(0, nmine, drain, 0)
    plsc.subcore_barrier()
    pltpu.sync_copy(deg_sh.at[pl.ds(s * TROW, TROW)],
                    deg_out.at[c, pl.ds(s * TROW, TROW)])


@functools.partial(
    pl.kernel,
    mesh=_mesh,
    out_type=jax.ShapeDtypeStruct((NCORE, NP, HALF), jnp.float32),
    scratch_types=[
        pltpu.VMEM((P0, EB), jnp.int32),
        pltpu.VMEM((P0, EB), jnp.int32),
        pltpu.VMEM((2, EB, HALF), jnp.float32),
        pltpu.VMEM_SHARED((NP, HALF), jnp.float32),
        pltpu.SemaphoreType.DMA((2,)),
    ],
)
def _msg_sc(tab3_hbm, e6_hbm, acc_out, sidx_v, didx_v, rows_v, acc_sh, gsem):
    c = lax.axis_index("c")
    s = lax.axis_index("s")

    def zfill(r, carry):
        for j in range(HALF // 16):
            rows_v[0, r, pl.ds(j * 16, 16)] = jnp.zeros((16,), jnp.float32)
        return carry

    lax.fori_loop(0, EB, zfill, 0)
    for k in range(TROW // EB):
        pltpu.sync_copy(rows_v.at[0],
                        acc_sh.at[pl.ds(s * TROW + k * EB, EB)])
    plsc.subcore_barrier()

    def body(b, nbp):
        slot = lax.rem(b, 2)
        nslot = lax.rem(b + 1, 2)

        @pl.when(b + 1 < nbp)
        def _():
            pltpu.async_copy(tab3_hbm.at[c].at[sidx_v.at[b + 1]],
                             rows_v.at[nslot], gsem.at[nslot])

        pltpu.make_async_copy(tab3_hbm.at[c].at[sidx_v.at[0]],
                              rows_v.at[slot], gsem.at[slot]).wait()
        pltpu.sync_copy(rows_v.at[slot], acc_sh.at[didx_v.at[b]], add=True)
        return nbp

    for pb in (0, P0):
        pltpu.sync_copy(e6_hbm.at[0, pl.ds(s * NBT + pb, P0)], sidx_v)
        pltpu.sync_copy(e6_hbm.at[1, pl.ds(s * NBT + pb, P0)], didx_v)
        pltpu.async_copy(tab3_hbm.at[c].at[sidx_v.at[0]], rows_v.at[0],
                         gsem.at[0])
        lax.fori_loop(0, P0, body, P0)
    plsc.subcore_barrier()
    pltpu.sync_copy(acc_sh.at[pl.ds(s * TROW, TROW)],
                    acc_out.at[c, pl.ds(s * TROW, TROW)])


_BN = 2560
_NBLK = NP // _BN


def _xws_body(x_ref, w_ref, deg_ref, tab_ref, dis_ref):
    xw = jnp.dot(x_ref[...].astype(jnp.bfloat16),
                 w_ref[...].astype(jnp.bfloat16),
                 preferred_element_type=jnp.float32)
    dt = jnp.transpose(deg_ref[...], (1, 0))
    dis = lax.rsqrt(dt[:, 0:1] + dt[:, 1:2] + 1.0)
    dis_ref[...] = dis
    tab_ref[0] = xw[:, :HALF] * dis
    tab_ref[1] = xw[:, HALF:] * dis


def _xws_tc(x, w, deg2):
    return pl.pallas_call(
        _xws_body,
        grid=(_NBLK,),
        in_specs=[pl.BlockSpec((_BN, DIN), lambda i: (i, 0)),
                  pl.BlockSpec((DIN, DHID), lambda i: (0, 0)),
                  pl.BlockSpec((NCORE, _BN), lambda i: (0, i))],
        out_specs=[pl.BlockSpec((NCORE, _BN, HALF), lambda i: (0, i, 0)),
                   pl.BlockSpec((_BN, 1), lambda i: (i, 0))],
        out_shape=[jax.ShapeDtypeStruct((NCORE, NP, HALF), jnp.float32),
                   jax.ShapeDtypeStruct((NP, 1), jnp.float32)],
    )(x, w, deg2)


def _epi_body(acc_ref, tab_ref, dis_ref, hn_ref, bg_ref, wf_ref, bf_ref,
              wc_ref, bc_ref, o_ref):
    dis = dis_ref[...]
    a = jnp.concatenate(
        [(acc_ref[0] + tab_ref[0]) * dis, (acc_ref[1] + tab_ref[1]) * dis],
        axis=1)
    h = jnp.maximum(a + bg_ref[...], 0.0)
    alpha = 1.0 - jnp.transpose(hn_ref[...], (1, 0))
    ha = alpha * h
    h2 = jnp.maximum(
        jnp.dot(ha, wf_ref[...], preferred_element_type=jnp.float32)
        + bf_ref[...], 0.0)
    out = (jnp.dot(h2, wc_ref[...], preferred_element_type=jnp.float32)
           + bc_ref[...])
    o_ref[...] = jnp.transpose(out, (1, 0))


def _epi_tc(acc, tab, dis, hn, bg, wf, bf, wc, bc):
    nc = wc.shape[1]
    return pl.pallas_call(
        _epi_body,
        grid=(_NBLK,),
        in_specs=[pl.BlockSpec((NCORE, _BN, HALF), lambda i: (0, i, 0)),
                  pl.BlockSpec((NCORE, _BN, HALF), lambda i: (0, i, 0)),
                  pl.BlockSpec((_BN, 1), lambda i: (i, 0)),
                  pl.BlockSpec((1, _BN), lambda i: (0, i)),
                  pl.BlockSpec((1, DHID), lambda i: (0, 0)),
                  pl.BlockSpec((DHID, 64), lambda i: (0, 0)),
                  pl.BlockSpec((1, 64), lambda i: (0, 0)),
                  pl.BlockSpec((64, nc), lambda i: (0, 0)),
                  pl.BlockSpec((1, nc), lambda i: (0, 0))],
        out_specs=pl.BlockSpec((nc, _BN), lambda i: (0, i)),
        out_shape=jax.ShapeDtypeStruct((nc, NN), jnp.float32),
    )(acc, tab, dis, hn, bg, wf, bf, wc, bc)


def kernel(x, edge_index, h_node, W_gcn, b_gcn, W_fus, b_fus, W_cls, b_cls):
    npad = NTPAD - NTILES
    ar = jnp.arange(npad * EB, dtype=jnp.int32).reshape(npad, EB)
    pad = jnp.stack([(ar * 89) % NN, NN + (ar % 16)])
    e6 = jnp.concatenate([edge_index.reshape(2, NTILES, EB), pad], axis=1)
    deg2 = _deg_sc(e6)
    tab, dis = _xws_tc(x, W_gcn, deg2)
    acc = _msg_sc(tab, e6)
    out_t = _epi_tc(acc, tab, dis, h_node.reshape(1, NN),
                    b_gcn.reshape(1, DHID), W_fus, b_fus.reshape(1, 64),
                    W_cls, b_cls.reshape(1, W_cls.shape[1]))
    return out_t.T


# speedup vs baseline: 1.1124x; 1.0061x over previous
import functools

import jax
import jax.numpy as jnp
from jax import lax
from jax.experimental import pallas as pl
from jax.experimental.pallas import tpu as pltpu
from jax.experimental.pallas import tpu_sc as plsc

NN = 10000
DIN = 256
DHID = 256
HALF = 128
NP = 10240
TROW = NP // 16
EB = 128
NTILES = 1250
NTPAD = 1280
NBT = NTPAD // 16
P0 = 40
NCORE = 2
NSUB = 16

_mesh = plsc.VectorSubcoreMesh(core_axis_name="c", subcore_axis_name="s")


@functools.partial(
    pl.kernel,
    mesh=_mesh,
    out_type=jax.ShapeDtypeStruct((NCORE, NP), jnp.float32),
    scratch_types=[
        pltpu.VMEM((NBT, EB), jnp.int32),
        pltpu.VMEM((EB,), jnp.float32),
        pltpu.VMEM((TROW,), jnp.float32),
        pltpu.VMEM_SHARED((NP,), jnp.float32),
        pltpu.SemaphoreType.DMA,
    ],
)
def _deg_sc(e6_hbm, deg_out, didx_v, ones_v, zbuf_v, deg_sh, sem):
    c = lax.axis_index("c")
    s = lax.axis_index("s")
    for i in range(EB // 16):
        ones_v[pl.ds(i * 16, 16)] = jnp.ones((16,), jnp.float32)
    for i in range(TROW // 16):
        zbuf_v[pl.ds(i * 16, 16)] = jnp.zeros((16,), jnp.float32)
    pltpu.sync_copy(e6_hbm.at[1, pl.ds(s * NBT, NBT)], didx_v)
    pltpu.sync_copy(zbuf_v, deg_sh.at[pl.ds(s * TROW, TROW)])
    plsc.subcore_barrier()

    nmine = NBT // 2

    def fire(b, carry):
        pltpu.async_copy(ones_v, deg_sh.at[didx_v.at[2 * b + c]], sem,
                         add=True)
        return carry

    lax.fori_loop(0, nmine, fire, 0)

    def drain(b, carry):
        pltpu.make_async_copy(ones_v, deg_sh.at[didx_v.at[0]], sem).wait()
        return carry

    lax.fori_loop(0, nmine, drain, 0)
    plsc.subcore_barrier()
    pltpu.sync_copy(deg_sh.at[pl.ds(s * TROW, TROW)],
                    deg_out.at[c, pl.ds(s * TROW, TROW)])


@functools.partial(
    pl.kernel,
    mesh=_mesh,
    out_type=jax.ShapeDtypeStruct((NCORE, NP, HALF), jnp.float32),
    scratch_types=[
        pltpu.VMEM((P0, EB), jnp.int32),
        pltpu.VMEM((P0, EB), jnp.int32),
        pltpu.VMEM((2, EB, HALF), jnp.float32),
        pltpu.VMEM_SHARED((NP, HALF), jnp.float32),
        pltpu.SemaphoreType.DMA((2,)),
    ],
)
def _msg_sc(tab3_hbm, e6_hbm, acc_out, sidx_v, didx_v, rows_v, acc_sh, gsem):
    c = lax.axis_index("c")
    s = lax.axis_index("s")

    def zfill(r, carry):
        for j in range(HALF // 16):
            rows_v[0, r, pl.ds(j * 16, 16)] = jnp.zeros((16,), jnp.float32)
        return carry

    lax.fori_loop(0, EB, zfill, 0)
    for k in range(TROW // EB):
        pltpu.sync_copy(rows_v.at[0],
                        acc_sh.at[pl.ds(s * TROW + k * EB, EB)])
    plsc.subcore_barrier()

    def body(b, nbp):
        slot = lax.rem(b, 2)
        nslot = lax.rem(b + 1, 2)

        @pl.when(b + 1 < nbp)
        def _():
            pltpu.async_copy(tab3_hbm.at[c].at[sidx_v.at[b + 1]],
                             rows_v.at[nslot], gsem.at[nslot])

        pltpu.make_async_copy(tab3_hbm.at[c].at[sidx_v.at[0]],
                              rows_v.at[slot], gsem.at[slot]).wait()
        pltpu.sync_copy(rows_v.at[slot], acc_sh.at[didx_v.at[b]], add=True)
        return nbp

    for pb in (0, P0):
        pltpu.sync_copy(e6_hbm.at[0, pl.ds(s * NBT + pb, P0)], sidx_v)
        pltpu.sync_copy(e6_hbm.at[1, pl.ds(s * NBT + pb, P0)], didx_v)
        pltpu.async_copy(tab3_hbm.at[c].at[sidx_v.at[0]], rows_v.at[0],
                         gsem.at[0])
        lax.fori_loop(0, P0, body, P0)
    plsc.subcore_barrier()
    pltpu.sync_copy(acc_sh.at[pl.ds(s * TROW, TROW)],
                    acc_out.at[c, pl.ds(s * TROW, TROW)])


_BN = 5120
_NBLK = NP // _BN


def _xws_body(x_ref, w_ref, deg_ref, tab_ref, dis_ref):
    xw = jnp.dot(x_ref[...].astype(jnp.bfloat16),
                 w_ref[...].astype(jnp.bfloat16),
                 preferred_element_type=jnp.float32)
    dt = jnp.transpose(deg_ref[...], (1, 0))
    dis = lax.rsqrt(dt[:, 0:1] + dt[:, 1:2] + 1.0)
    dis_ref[...] = dis
    tab_ref[0] = xw[:, :HALF] * dis
    tab_ref[1] = xw[:, HALF:] * dis


def _xws_tc(x, w, deg2):
    return pl.pallas_call(
        _xws_body,
        grid=(_NBLK,),
        in_specs=[pl.BlockSpec((_BN, DIN), lambda i: (i, 0)),
                  pl.BlockSpec((DIN, DHID), lambda i: (0, 0)),
                  pl.BlockSpec((NCORE, _BN), lambda i: (0, i))],
        out_specs=[pl.BlockSpec((NCORE, _BN, HALF), lambda i: (0, i, 0)),
                   pl.BlockSpec((_BN, 1), lambda i: (i, 0))],
        out_shape=[jax.ShapeDtypeStruct((NCORE, NP, HALF), jnp.float32),
                   jax.ShapeDtypeStruct((NP, 1), jnp.float32)],
    )(x, w, deg2)


def _epi_body(acc_ref, tab_ref, dis_ref, hn_ref, bg_ref, wf_ref, bf_ref,
              wc_ref, bc_ref, o_ref):
    dis = dis_ref[...]
    a = jnp.concatenate(
        [(acc_ref[0] + tab_ref[0]) * dis, (acc_ref[1] + tab_ref[1]) * dis],
        axis=1)
    h = jnp.maximum(a + bg_ref[...], 0.0)
    alpha = 1.0 - jnp.transpose(hn_ref[...], (1, 0))
    ha = alpha * h
    h2 = jnp.maximum(
        jnp.dot(ha, wf_ref[...], preferred_element_type=jnp.float32)
        + bf_ref[...], 0.0)
    out = (jnp.dot(h2, wc_ref[...], preferred_element_type=jnp.float32)
           + bc_ref[...])
    o_ref[...] = jnp.transpose(out, (1, 0))


def _epi_tc(acc, tab, dis, hn, bg, wf, bf, wc, bc):
    nc = wc.shape[1]
    return pl.pallas_call(
        _epi_body,
        grid=(_NBLK,),
        in_specs=[pl.BlockSpec((NCORE, _BN, HALF), lambda i: (0, i, 0)),
                  pl.BlockSpec((NCORE, _BN, HALF), lambda i: (0, i, 0)),
                  pl.BlockSpec((_BN, 1), lambda i: (i, 0)),
                  pl.BlockSpec((1, _BN), lambda i: (0, i)),
                  pl.BlockSpec((1, DHID), lambda i: (0, 0)),
                  pl.BlockSpec((DHID, 64), lambda i: (0, 0)),
                  pl.BlockSpec((1, 64), lambda i: (0, 0)),
                  pl.BlockSpec((64, nc), lambda i: (0, 0)),
                  pl.BlockSpec((1, nc), lambda i: (0, 0))],
        out_specs=pl.BlockSpec((nc, _BN), lambda i: (0, i)),
        out_shape=jax.ShapeDtypeStruct((nc, NN), jnp.float32),
    )(acc, tab, dis, hn, bg, wf, bf, wc, bc)


def kernel(x, edge_index, h_node, W_gcn, b_gcn, W_fus, b_fus, W_cls, b_cls):
    npad = NTPAD - NTILES
    ar = jnp.arange(npad * EB, dtype=jnp.int32).reshape(npad, EB)
    pad = jnp.stack([(ar * 89) % NN, NN + (ar % 16)])
    e6 = jnp.concatenate([edge_index.reshape(2, NTILES, EB), pad], axis=1)
    deg2 = _deg_sc(e6)
    tab, dis = _xws_tc(x, W_gcn, deg2)
    acc = _msg_sc(tab, e6)
    out_t = _epi_tc(acc, tab, dis, h_node.reshape(1, NN),
                    b_gcn.reshape(1, DHID), W_fus, b_fus.reshape(1, 64),
                    W_cls, b_cls.reshape(1, W_cls.shape[1]))
    return out_t.T
